# baseline (speedup 1.0000x reference)
import functools

import jax
import jax.numpy as jnp
from jax import lax
from jax.experimental import pallas as pl
from jax.experimental.pallas import tpu as pltpu
from jax.experimental.pallas import tpu_sc as plsc

NUM_CORES = 2
NUM_SUBCORES = 16
NW = NUM_CORES * NUM_SUBCORES
CBW = 65536
STRIPE = CBW // 8
CHUNK = 128


def _precompute_body(*refs):
    xs, wb_ref, o_ref = refs[:8], refs[8], refs[9]
    x = jnp.concatenate([r[...] for r in xs], axis=0)
    o_ref[...] = jax.lax.dot_general(
        x.astype(jnp.bfloat16), wb_ref[...].astype(jnp.bfloat16),
        dimension_numbers=(((0,), (0,)), ((), ())),
        preferred_element_type=jnp.float32)


def _precompute_packed(table_t, w):
    n = table_t.shape[1]
    nblk = pl.cdiv(n, CBW)
    wb = jnp.kron(jnp.eye(8, dtype=w.dtype), w)
    last = pl.cdiv(n, STRIPE) - 1
    stripe_specs = [
        pl.BlockSpec((32, STRIPE),
                     lambda i, j=j: (0, jnp.minimum(i * 8 + j, last)))
        for j in range(8)
    ]
    return pl.pallas_call(
        _precompute_body,
        grid=(nblk,),
        in_specs=stripe_specs + [pl.BlockSpec((256, 128), lambda i: (0, 0))],
        out_specs=pl.BlockSpec((STRIPE, 128), lambda i: (i, 0)),
        out_shape=jax.ShapeDtypeStruct((nblk * STRIPE, 128), jnp.float32),
    )(*([table_t] * 8), wb)


def _sc_gather_mlp(user_id, item_id, price, t_u, t_i, params):
    B = user_id.shape[0]
    bpw = B // NW
    nchunk = bpw // CHUNK
    mesh = plsc.VectorSubcoreMesh(core_axis_name="c", subcore_axis_name="s")

    @functools.partial(
        pl.kernel,
        mesh=mesh,
        compiler_params=pltpu.CompilerParams(needs_layout_passes=False,
                                             use_tc_tiling_on_sc=False),
        out_type=jax.ShapeDtypeStruct((B,), jnp.float32),
        scratch_types=[
            pltpu.VMEM((bpw,), jnp.int32),
            pltpu.VMEM((bpw,), jnp.int32),
            pltpu.VMEM((bpw,), jnp.float32),
            pltpu.VMEM((nchunk, CHUNK), jnp.int32),
            pltpu.VMEM((nchunk, CHUNK), jnp.int32),
        ] + [pltpu.VMEM((CHUNK, 16), jnp.float32)] * 8 + [
            pltpu.VMEM((bpw,), jnp.float32),
            pltpu.VMEM((64,), jnp.float32),
        ] + [pltpu.SemaphoreType.DMA] * 8,
    )
    def body(uid_hbm, iid_hbm, price_hbm, tu_hbm, ti_hbm, par_hbm, out_hbm,
             uid_v, iid_v, pr_v, gu_v, gi_v,
             su0, su1, su2, su3, si0, si1, si2, si3, z_v, par_v,
             mu0, mu1, mu2, mu3, mi0, mi1, mi2, mi3):
        wid = lax.axis_index("s") * NUM_CORES + lax.axis_index("c")
        base = wid * bpw
        pltpu.sync_copy(uid_hbm.at[pl.ds(base, bpw)], uid_v)
        pltpu.sync_copy(iid_hbm.at[pl.ds(base, bpw)], iid_v)
        pltpu.sync_copy(price_hbm.at[pl.ds(base, bpw)], pr_v)
        pltpu.sync_copy(par_hbm, par_v)
        iota = lax.iota(jnp.int32, 16)
        w1p_s = [plsc.load_gather(par_v, [jnp.full((16,), k, jnp.int32)])
                 for k in range(16)]
        b1_s = [plsc.load_gather(par_v, [jnp.full((16,), 16 + k, jnp.int32)])
                for k in range(16)]
        w2_s = [plsc.load_gather(par_v, [jnp.full((16,), 32 + k, jnp.int32)])
                for k in range(16)]
        b2 = par_v[pl.ds(48, 16)]

        for c in range(nchunk):
            for s in range(CHUNK // 16):
                u = uid_v[pl.ds(c * CHUNK + s * 16, 16)]
                it = iid_v[pl.ds(c * CHUNK + s * 16, 16)]
                gu_v[c, pl.ds(s * 16, 16)] = (
                    (lax.shift_right_logical(u, 16) * 8192 + (u & 8191)) * 8
                    + (lax.shift_right_logical(u, 13) & 7))
                gi_v[c, pl.ds(s * 16, 16)] = (
                    (lax.shift_right_logical(it, 16) * 8192 + (it & 8191)) * 8
                    + (lax.shift_right_logical(it, 13) & 7))

        subufs = [su0, su1, su2, su3]
        sibufs = [si0, si1, si2, si3]
        musems = [mu0, mu1, mu2, mu3]
        misems = [mi0, mi1, mi2, mi3]

        pend = [(pltpu.async_copy(tu_hbm.at[gu_v.at[c]], subufs[c], musems[c]),
                 pltpu.async_copy(ti_hbm.at[gi_v.at[c]], sibufs[c], misems[c]))
                for c in range(nchunk)]
        for c in range(nchunk):
            pend[c][0].wait()
            pend[c][1].wait()
            su, si = subufs[c], sibufs[c]
            for g in range(CHUNK // 16):
                off = c * CHUNK + g * 16
                ridx = iota + g * 16
                p16 = pr_v[pl.ds(off, 16)]
                acc = jnp.zeros((16,), jnp.float32)
                for k in range(16):
                    kidx = jnp.full((16,), k, jnp.int32)
                    hk = (plsc.load_gather(su, [ridx, kidx])
                          + plsc.load_gather(si, [ridx, kidx])
                          + p16 * w1p_s[k] + b1_s[k])
                    acc = acc + jnp.maximum(hk, 0.0) * w2_s[k]
                z_v[pl.ds(off, 16)] = 1.0 / (1.0 + jnp.exp(-acc - b2))

        pltpu.sync_copy(z_v, out_hbm.at[pl.ds(base, bpw)])

    return body(user_id, item_id, price, t_u, t_i, params)


def kernel(user_id, item_id, price, user_table, item_table, W1, b1, W2, b2):
    D = user_table.shape[1]
    t_u = _precompute_packed(user_table.T, W1[0:D, :]).reshape(-1, 16)
    t_i = _precompute_packed(item_table.T, W1[D:2 * D, :]).reshape(-1, 16)
    params = jnp.concatenate(
        [W1[2 * D, :], b1, W2[:, 0], jnp.full((16,), b2[0], jnp.float32)])
    return _sc_gather_mlp(user_id, item_id, price, t_u, t_i, params)
